# baseline (speedup 1.0000x reference)
import dataclasses
import functools

import jax
import jax.numpy as jnp
from jax import lax
from jax.experimental import pallas as pl
from jax.experimental.pallas import tpu as pltpu
from jax.experimental.pallas import tpu_sc as plsc

_N = 10000
_E = 160000
_C = 128
_NRBF = 16
_CUTOFF = 5.0
_GAMMA = 10.0
_NORM = 16.0

_NC = 2
_NS = 16
_NG = 4
_GC = _C // _NG

_EPT1 = _E // (_NC * _NS)
_EPT1_PAD = _EPT1 + 8
_EPT = _E // _NS
_B = 80
_NP = 10240
_RPT = _NP // _NS

_mesh = plsc.VectorSubcoreMesh(
    core_axis_name="c", subcore_axis_name="s", num_cores=_NC, num_subcores=_NS)

_sc_params = pltpu.CompilerParams()
if "needs_layout_passes" in pltpu.CompilerParams.__dataclass_fields__:
    _sc_params = dataclasses.replace(_sc_params, needs_layout_passes=False)


_RCH = 1000
_RCHP = _RCH + 8


@functools.partial(
    pl.kernel,
    out_type=jax.ShapeDtypeStruct((4 * _E,), jnp.float32),
    mesh=_mesh,
    compiler_params=_sc_params,
    scratch_types=[
        pltpu.VMEM((4 * _N,), jnp.float32),
        pltpu.VMEM((_RCHP,), jnp.int32),
        pltpu.VMEM((_RCHP,), jnp.int32),
        pltpu.VMEM((4 * _RCHP,), jnp.float32),
    ],
)
def _rij_kernel(coords_hbm, ii_hbm, jj_hbm, rij_hbm, cbuf, ibuf, jbuf, rbuf):
    wid = lax.axis_index("s") * _NC + lax.axis_index("c")
    base = wid * _EPT1
    pltpu.sync_copy(coords_hbm, cbuf)
    ibuf[pl.ds(_RCHP - 16, 16)] = jnp.zeros((16,), jnp.int32)
    jbuf[pl.ds(_RCHP - 16, 16)] = jnp.zeros((16,), jnp.int32)
    lanes = jnp.arange(16, dtype=jnp.int32)

    @pl.loop(0, _EPT1, step=_RCH)
    def _(cb):
        pltpu.sync_copy(ii_hbm.at[pl.ds(base + cb, _RCH)],
                        ibuf.at[pl.ds(0, _RCH)])
        pltpu.sync_copy(jj_hbm.at[pl.ds(base + cb, _RCH)],
                        jbuf.at[pl.ds(0, _RCH)])

        @pl.loop(0, _RCHP, step=16)
        def _(e):
            vi = ibuf[pl.ds(e, 16)] * 4
            vj = jbuf[pl.ds(e, 16)] * 4
            dst = (e + lanes) * 4
            for x in range(3):
                cix = plsc.load_gather(cbuf, [vi + x])
                cjx = plsc.load_gather(cbuf, [vj + x])
                plsc.store_scatter(rbuf, [dst + x], cjx - cix)

        pltpu.sync_copy(rbuf.at[pl.ds(0, 4 * _RCH)],
                        rij_hbm.at[pl.ds(4 * (base + cb), 4 * _RCH)])


_BE = 2000


def _edge_feat_body(rij_ref, w_ref, fn_ref, u_ref):
    r = rij_ref[...]
    r3 = r[:, 0:3]
    d2 = jnp.sum(r3 * r3, axis=1, keepdims=True) + 1e-12
    inv_d = lax.rsqrt(d2)
    d = d2 * inv_d
    u_ref[...] = r * inv_d
    centers = (lax.broadcasted_iota(jnp.int32, (1, _NRBF), 1)
               .astype(jnp.float32) * (_CUTOFF / (_NRBF - 1)))
    rbf = jnp.exp(-_GAMMA * (d - centers) ** 2) * (1.0 / _NORM)
    fn = lax.dot_general(rbf, w_ref[...],
                         (((1,), (1,)), ((), ())),
                         preferred_element_type=jnp.float32,
                         precision=lax.Precision.HIGHEST)
    fn_ref[...] = fn.reshape(_BE, _NG, _C)


_EH = _E // 2
_NBLK = _EH // _BE


def _edge_features(rij, w_all, half):
    off = half * _NBLK
    return pl.pallas_call(
        _edge_feat_body,
        grid=(_NBLK,),
        in_specs=[
            pl.BlockSpec((_BE, 4), lambda i: (i + off, 0)),
            pl.BlockSpec((_NG * _C, _NRBF), lambda i: (0, 0)),
        ],
        out_specs=[
            pl.BlockSpec((_BE, _NG, _C), lambda i: (i, 0, 0)),
            pl.BlockSpec((_BE, 4), lambda i: (i, 0)),
        ],
        out_shape=[
            jax.ShapeDtypeStruct((_EH, _NG, _C), jnp.float32),
            jax.ShapeDtypeStruct((_EH, 4), jnp.float32),
        ],
    )(rij, w_all)


_B = 40
_EPTH = _EH // _NS
_CPTH = _EPTH // _B


def _make_agg(eoff, carry):
    @functools.partial(
        pl.kernel,
        out_type=jax.ShapeDtypeStruct((_NG, _NP, _C), jnp.float32),
        mesh=_mesh,
        compiler_params=_sc_params,
        scratch_types=[
            pltpu.VMEM_SHARED((_NP, _C), jnp.float32),
            pltpu.VMEM((4, 1, _B), jnp.int32),
            pltpu.VMEM((4, 1, _B), jnp.int32),
            pltpu.VMEM((2 * 4 * _B,), jnp.float32),
            pltpu.VMEM((2, _B, _C), jnp.float32),
            pltpu.VMEM((2, _B, _C), jnp.float32),
            pltpu.VMEM((2, _B, _C), jnp.float32),
            pltpu.SemaphoreType.DMA((4,)),
            pltpu.SemaphoreType.DMA((4,)),
            pltpu.SemaphoreType.DMA((2,)),
            pltpu.SemaphoreType.DMA((2,)),
            pltpu.SemaphoreType.DMA((2,)),
            pltpu.SemaphoreType.DMA((2,)),
        ],
    )
    def _agg(x_hbm, fn_hbm, u_hbm, ii_hbm, jj_hbm, init_hbm, out_hbm,
             acc, iibuf, jjbuf, ubuf, xbuf, fnbuf, mbuf,
             sem_ii, sem_jj, sem_x, sem_fn, sem_u, sem_sc):
        c = lax.axis_index("c")
        s = lax.axis_index("s")
        rows0 = s * _RPT

        for p in range(2):
            g = c * 2 + p

            def issue_ii(k, q):
                pltpu.async_copy(ii_hbm.at[pl.ds(eoff + s * _EPTH + k * _B, _B)],
                                 iibuf.at[q, 0], sem_ii.at[q])

            def wait_ii(k, q):
                pltpu.make_async_copy(
                    ii_hbm.at[pl.ds(eoff + s * _EPTH + k * _B, _B)],
                    iibuf.at[q, 0], sem_ii.at[q]).wait()

            def issue_jj(k, q):
                pltpu.async_copy(jj_hbm.at[pl.ds(eoff + s * _EPTH + k * _B, _B)],
                                 jjbuf.at[q, 0], sem_jj.at[q])

            def wait_jj(k, q):
                pltpu.make_async_copy(
                    jj_hbm.at[pl.ds(eoff + s * _EPTH + k * _B, _B)],
                    jjbuf.at[q, 0], sem_jj.at[q]).wait()

            def issue(k, b, q):
                base = s * _EPTH + k * _B
                pltpu.async_copy(x_hbm.at[g].at[jjbuf.at[q, 0]], xbuf.at[b],
                                 sem_x.at[b])
                pltpu.async_copy(fn_hbm.at[pl.ds(base, _B), g], fnbuf.at[b],
                                 sem_fn.at[b])
                pltpu.async_copy(u_hbm.at[pl.ds(4 * base, 4 * _B)],
                                 ubuf.at[pl.ds(b * 4 * _B, 4 * _B)],
                                 sem_u.at[b])

            def wait_in(k, b, q):
                base = s * _EPTH + k * _B
                pltpu.make_async_copy(x_hbm.at[g].at[jjbuf.at[q, 0]],
                                      xbuf.at[b], sem_x.at[b]).wait()
                pltpu.make_async_copy(fn_hbm.at[pl.ds(base, _B), g],
                                      fnbuf.at[b], sem_fn.at[b]).wait()
                pltpu.make_async_copy(u_hbm.at[pl.ds(4 * base, 4 * _B)],
                                      ubuf.at[pl.ds(b * 4 * _B, 4 * _B)],
                                      sem_u.at[b]).wait()

            def scat(k, b, q):
                pltpu.async_copy(mbuf.at[b], acc.at[iibuf.at[q, 0]],
                                 sem_sc.at[b], add=True)

            def wait_sc(k, b, q):
                pltpu.make_async_copy(mbuf.at[b], acc.at[iibuf.at[q, 0]],
                                      sem_sc.at[b]).wait()

            def compute(k, b):
                uoff = b * (4 * _B)

                @plsc.parallel_loop(0, _B, 1, unroll=10)
                def _(e):
                    ub = uoff + 4 * e
                    ux = plsc.load_gather(ubuf, [jnp.full((16,), ub, jnp.int32)])
                    uy = plsc.load_gather(ubuf, [jnp.full((16,), ub + 1, jnp.int32)])
                    uz = plsc.load_gather(ubuf, [jnp.full((16,), ub + 2, jnp.int32)])
                    for h in range(2):
                        o = 16 * h
                        x0v = xbuf[b, e, pl.ds(o, 16)]
                        x1x = xbuf[b, e, pl.ds(32 + o, 16)]
                        x1y = xbuf[b, e, pl.ds(64 + o, 16)]
                        x1z = xbuf[b, e, pl.ds(96 + o, 16)]
                        f000 = fnbuf[b, e, pl.ds(o, 16)]
                        f011 = fnbuf[b, e, pl.ds(32 + o, 16)]
                        f101 = fnbuf[b, e, pl.ds(64 + o, 16)]
                        f110 = fnbuf[b, e, pl.ds(96 + o, 16)]
                        xdu = x1x * ux + x1y * uy + x1z * uz
                        t = x0v * f011
                        mbuf[b, e, pl.ds(o, 16)] = x0v * f000 + f110 * xdu
                        mbuf[b, e, pl.ds(32 + o, 16)] = t * ux + x1x * f101
                        mbuf[b, e, pl.ds(64 + o, 16)] = t * uy + x1y * f101
                        mbuf[b, e, pl.ds(96 + o, 16)] = t * uz + x1z * f101

            def step(k, b, q):
                @pl.when(k >= 2)
                def _():
                    wait_sc(k - 2, b, (q + 2) % 4)
                wait_in(k, b, q)
                compute(k, b)
                wait_ii(k, q)
                scat(k, b, q)

                @pl.when(k + 2 <= _CPTH - 1)
                def _():
                    wait_jj(k + 2, (q + 2) % 4)
                    issue(k + 2, b, (q + 2) % 4)
                    issue_ii(k + 2, (q + 2) % 4)

                @pl.when(k + 3 <= _CPTH - 1)
                def _():
                    issue_jj(k + 3, (q + 3) % 4)

            if carry:
                pltpu.sync_copy(init_hbm.at[g, pl.ds(rows0, _RPT)],
                                acc.at[pl.ds(rows0, _RPT)])
            else:
                pltpu.sync_copy(init_hbm, acc.at[pl.ds(rows0, _RPT)])
            plsc.subcore_barrier()
            issue_jj(0, 0)
            issue_jj(1, 1)
            issue_jj(2, 2)
            issue_ii(0, 0)
            issue_ii(1, 1)
            wait_jj(0, 0)
            issue(0, 0, 0)
            wait_jj(1, 1)
            issue(1, 1, 1)

            @pl.loop(0, _CPTH - 1, step=4)
            def _(k):
                step(k, 0, 0)
                step(k + 1, 1, 1)
                step(k + 2, 0, 2)
                step(k + 3, 1, 3)

            step(_CPTH - 1, 0, 0)
            wait_sc(_CPTH - 2, 1, 3)
            wait_sc(_CPTH - 1, 0, 0)
            plsc.subcore_barrier()
            pltpu.sync_copy(acc.at[pl.ds(rows0, _RPT)],
                            out_hbm.at[g, pl.ds(rows0, _RPT)])
            plsc.subcore_barrier()

    return _agg


_agg_first = _make_agg(0, False)
_agg_second = _make_agg(_EH, True)


_NB = 400


def _final_body(out_ref, w0_ref, b0_ref, w1_ref,
                aw0_ref, ab0_ref, aw1_ref, ab1_ref,
                y0_ref, yx_ref, yy_ref, yz_ref):
    t = out_ref[...]

    def sec(s):
        return (t[:, :, s * _GC:(s + 1) * _GC]
                .transpose(1, 0, 2).reshape(_NB, _C))

    dn = (((1,), (1,)), ((), ()))
    kw = dict(preferred_element_type=jnp.float32,
              precision=lax.Precision.HIGHEST)
    h0 = lax.dot_general(sec(0), w0_ref[...], dn, **kw) + b0_ref[...]
    hx = lax.dot_general(sec(1), w1_ref[...], dn, **kw)
    hy = lax.dot_general(sec(2), w1_ref[...], dn, **kw)
    hz = lax.dot_general(sec(3), w1_ref[...], dn, **kw)
    y0_ref[...] = h0 * jax.nn.sigmoid(aw0_ref[...] * h0 + ab0_ref[...])
    nrm2 = hx * hx + hy * hy + hz * hz + 1e-12
    nrm = nrm2 * lax.rsqrt(nrm2)
    a = aw1_ref[...] * nrm + ab1_ref[...]
    gate = a * jax.nn.sigmoid(a)
    yx_ref[...] = hx * gate
    yy_ref[...] = hy * gate
    yz_ref[...] = hz * gate


def _final(out, w0, b0, w1, aw0, ab0, aw1, ab1):
    row = pl.BlockSpec((_NB, _C), lambda i: (i, 0))
    full = pl.BlockSpec((_C, _C), lambda i: (0, 0))
    vec = pl.BlockSpec((1, _C), lambda i: (0, 0))
    return pl.pallas_call(
        _final_body,
        grid=(_N // _NB,),
        in_specs=[pl.BlockSpec((_NG, _NB, _C), lambda i: (0, i, 0)),
                  full, vec, full, vec, vec, vec, vec],
        out_specs=[row, row, row, row],
        out_shape=[jax.ShapeDtypeStruct((_N, _C), jnp.float32)] * 4,
    )(out, w0, b0, w1, aw0, ab0, aw1, ab1)


def kernel(x0, x1, coordinates, W_rbf_000, W_rbf_011, W_rbf_101, W_rbf_110,
           W_si0, b_si0, W_si1, act_w0, act_b0, act_w1, act_b1,
           edge_index, atomic_number):
    idx_i = edge_index[0]
    idx_j = edge_index[1]
    coords_flat = jnp.pad(coordinates, ((0, 0), (0, 1))).reshape(-1)

    x1t = x1.transpose(2, 0, 1)
    parts = jnp.stack([x0.reshape(_N, _NG, _GC),
                       x1t[0].reshape(_N, _NG, _GC),
                       x1t[1].reshape(_N, _NG, _GC),
                       x1t[2].reshape(_N, _NG, _GC)], axis=2)
    xp = parts.transpose(1, 0, 2, 3).reshape(_NG, _N, _C)
    rij = _rij_kernel(coords_flat, idx_i, idx_j).reshape(_E, 4)

    w_all = (jnp.stack([W_rbf_000, W_rbf_011, W_rbf_101, W_rbf_110], 0)
             .reshape(_NG, _NG, _GC, _NRBF)
             .transpose(1, 0, 2, 3)
             .reshape(_NG * _C, _NRBF))
    fn1, u1 = _edge_features(rij, w_all, 0)
    fn2, u2 = _edge_features(rij, w_all, 1)

    z = jnp.zeros((_RPT, _C), jnp.float32)
    out1 = _agg_first(xp, fn1, u1.reshape(-1), idx_i, idx_j, z)
    out = _agg_second(xp, fn2, u2.reshape(-1), idx_i, idx_j, out1)

    y0, yx, yy, yz = _final(
        out, W_si0, b_si0.reshape(1, _C), W_si1,
        act_w0.reshape(1, _C), act_b0.reshape(1, _C),
        act_w1.reshape(1, _C), act_b1.reshape(1, _C))
    y1 = jnp.stack([yx, yy, yz], axis=-1)
    return (y0, y1)


# speedup vs baseline: 1.0073x; 1.0073x over previous
import dataclasses
import functools

import jax
import jax.numpy as jnp
from jax import lax
from jax.experimental import pallas as pl
from jax.experimental.pallas import tpu as pltpu
from jax.experimental.pallas import tpu_sc as plsc

_N = 10000
_E = 160000
_C = 128
_NRBF = 16
_CUTOFF = 5.0
_GAMMA = 10.0
_NORM = 16.0

_NC = 2
_NS = 16
_NG = 4
_GC = _C // _NG

_EPT1 = _E // (_NC * _NS)
_EPT1_PAD = _EPT1 + 8
_EPT = _E // _NS
_B = 80
_NP = 10240
_RPT = _NP // _NS

_mesh = plsc.VectorSubcoreMesh(
    core_axis_name="c", subcore_axis_name="s", num_cores=_NC, num_subcores=_NS)

_sc_params = pltpu.CompilerParams()
if "needs_layout_passes" in pltpu.CompilerParams.__dataclass_fields__:
    _sc_params = dataclasses.replace(_sc_params, needs_layout_passes=False)


_RCH = 1000
_RCHP = _RCH + 8


@functools.partial(
    pl.kernel,
    out_type=jax.ShapeDtypeStruct((4 * _E,), jnp.float32),
    mesh=_mesh,
    compiler_params=_sc_params,
    scratch_types=[
        pltpu.VMEM((4 * _N,), jnp.float32),
        pltpu.VMEM((_RCHP,), jnp.int32),
        pltpu.VMEM((_RCHP,), jnp.int32),
        pltpu.VMEM((4 * _RCHP,), jnp.float32),
    ],
)
def _rij_kernel(coords_hbm, ii_hbm, jj_hbm, rij_hbm, cbuf, ibuf, jbuf, rbuf):
    wid = lax.axis_index("s") * _NC + lax.axis_index("c")
    base = wid * _EPT1
    pltpu.sync_copy(coords_hbm, cbuf)
    ibuf[pl.ds(_RCHP - 16, 16)] = jnp.zeros((16,), jnp.int32)
    jbuf[pl.ds(_RCHP - 16, 16)] = jnp.zeros((16,), jnp.int32)
    lanes = jnp.arange(16, dtype=jnp.int32)

    @pl.loop(0, _EPT1, step=_RCH)
    def _(cb):
        pltpu.sync_copy(ii_hbm.at[pl.ds(base + cb, _RCH)],
                        ibuf.at[pl.ds(0, _RCH)])
        pltpu.sync_copy(jj_hbm.at[pl.ds(base + cb, _RCH)],
                        jbuf.at[pl.ds(0, _RCH)])

        @pl.loop(0, _RCHP, step=16)
        def _(e):
            vi = ibuf[pl.ds(e, 16)] * 4
            vj = jbuf[pl.ds(e, 16)] * 4
            dst = (e + lanes) * 4
            for x in range(3):
                cix = plsc.load_gather(cbuf, [vi + x])
                cjx = plsc.load_gather(cbuf, [vj + x])
                plsc.store_scatter(rbuf, [dst + x], cjx - cix)

        pltpu.sync_copy(rbuf.at[pl.ds(0, 4 * _RCH)],
                        rij_hbm.at[pl.ds(4 * (base + cb), 4 * _RCH)])


_BE = 2000


def _edge_feat_body(rij_ref, w_ref, fn_ref, u_ref):
    r = rij_ref[...]
    r3 = r[:, 0:3]
    d2 = jnp.sum(r3 * r3, axis=1, keepdims=True) + 1e-12
    inv_d = lax.rsqrt(d2)
    d = d2 * inv_d
    u_ref[...] = r * inv_d
    centers = (lax.broadcasted_iota(jnp.int32, (1, _NRBF), 1)
               .astype(jnp.float32) * (_CUTOFF / (_NRBF - 1)))
    rbf = jnp.exp(-_GAMMA * (d - centers) ** 2) * (1.0 / _NORM)
    fn = lax.dot_general(rbf, w_ref[...],
                         (((1,), (1,)), ((), ())),
                         preferred_element_type=jnp.float32,
                         precision=lax.Precision.HIGHEST)
    fn_ref[...] = fn.reshape(_BE, _NG, _C)


_EH = _E // 2
_NBLK = _EH // _BE


def _edge_features(rij, w_all, half):
    off = half * _NBLK
    return pl.pallas_call(
        _edge_feat_body,
        grid=(_NBLK,),
        in_specs=[
            pl.BlockSpec((_BE, 4), lambda i: (i + off, 0)),
            pl.BlockSpec((_NG * _C, _NRBF), lambda i: (0, 0)),
        ],
        out_specs=[
            pl.BlockSpec((_BE, _NG, _C), lambda i: (i, 0, 0)),
            pl.BlockSpec((_BE, 4), lambda i: (i, 0)),
        ],
        out_shape=[
            jax.ShapeDtypeStruct((_EH, _NG, _C), jnp.float32),
            jax.ShapeDtypeStruct((_EH, 4), jnp.float32),
        ],
    )(rij, w_all)


_B = 40
_EPTH = _EH // _NS
_CPTH = _EPTH // _B


def _make_agg(eoff, carry):
    @functools.partial(
        pl.kernel,
        out_type=jax.ShapeDtypeStruct((_NG, _NP, _C), jnp.float32),
        mesh=_mesh,
        compiler_params=_sc_params,
        scratch_types=[
            pltpu.VMEM_SHARED((_NP, _C), jnp.float32),
            pltpu.VMEM((4, 1, _B), jnp.int32),
            pltpu.VMEM((4, 1, _B), jnp.int32),
            pltpu.VMEM((2 * 4 * _B,), jnp.float32),
            pltpu.VMEM((2, _B, _C), jnp.float32),
            pltpu.VMEM((2, _B, _C), jnp.float32),
            pltpu.VMEM((2, _B, _C), jnp.float32),
            pltpu.SemaphoreType.DMA((4,)),
            pltpu.SemaphoreType.DMA((4,)),
            pltpu.SemaphoreType.DMA((2,)),
            pltpu.SemaphoreType.DMA((2,)),
            pltpu.SemaphoreType.DMA((2,)),
            pltpu.SemaphoreType.DMA((2,)),
        ],
    )
    def _agg(x_hbm, fn_hbm, u_hbm, ii_hbm, jj_hbm, init_hbm, out_hbm,
             acc, iibuf, jjbuf, ubuf, xbuf, fnbuf, mbuf,
             sem_ii, sem_jj, sem_x, sem_fn, sem_u, sem_sc):
        c = lax.axis_index("c")
        s = lax.axis_index("s")
        rows0 = s * _RPT

        for p in range(2):
            g = c * 2 + p

            def issue_ii(k, q):
                pltpu.async_copy(ii_hbm.at[pl.ds(eoff + s * _EPTH + k * _B, _B)],
                                 iibuf.at[q, 0], sem_ii.at[q])

            def wait_ii(k, q):
                pltpu.make_async_copy(
                    ii_hbm.at[pl.ds(eoff + s * _EPTH + k * _B, _B)],
                    iibuf.at[q, 0], sem_ii.at[q]).wait()

            def issue_jj(k, q):
                pltpu.async_copy(jj_hbm.at[pl.ds(eoff + s * _EPTH + k * _B, _B)],
                                 jjbuf.at[q, 0], sem_jj.at[q])

            def wait_jj(k, q):
                pltpu.make_async_copy(
                    jj_hbm.at[pl.ds(eoff + s * _EPTH + k * _B, _B)],
                    jjbuf.at[q, 0], sem_jj.at[q]).wait()

            def issue(k, b, q):
                base = s * _EPTH + k * _B
                pltpu.async_copy(x_hbm.at[g].at[jjbuf.at[q, 0]], xbuf.at[b],
                                 sem_x.at[b])
                pltpu.async_copy(fn_hbm.at[pl.ds(base, _B), g], fnbuf.at[b],
                                 sem_fn.at[b])
                pltpu.async_copy(u_hbm.at[pl.ds(4 * base, 4 * _B)],
                                 ubuf.at[pl.ds(b * 4 * _B, 4 * _B)],
                                 sem_u.at[b])

            def wait_in(k, b, q):
                base = s * _EPTH + k * _B
                pltpu.make_async_copy(x_hbm.at[g].at[jjbuf.at[q, 0]],
                                      xbuf.at[b], sem_x.at[b]).wait()
                pltpu.make_async_copy(fn_hbm.at[pl.ds(base, _B), g],
                                      fnbuf.at[b], sem_fn.at[b]).wait()
                pltpu.make_async_copy(u_hbm.at[pl.ds(4 * base, 4 * _B)],
                                      ubuf.at[pl.ds(b * 4 * _B, 4 * _B)],
                                      sem_u.at[b]).wait()

            def scat(k, b, q):
                pltpu.async_copy(mbuf.at[b], acc.at[iibuf.at[q, 0]],
                                 sem_sc.at[b], add=True)

            def wait_sc(k, b, q):
                pltpu.make_async_copy(mbuf.at[b], acc.at[iibuf.at[q, 0]],
                                      sem_sc.at[b]).wait()

            def compute(k, b):
                uoff = b * (4 * _B)

                @plsc.parallel_loop(0, _B, 1, unroll=8)
                def _(e):
                    ub = uoff + 4 * e
                    ux = plsc.load_gather(ubuf, [jnp.full((16,), ub, jnp.int32)])
                    uy = plsc.load_gather(ubuf, [jnp.full((16,), ub + 1, jnp.int32)])
                    uz = plsc.load_gather(ubuf, [jnp.full((16,), ub + 2, jnp.int32)])
                    for h in range(2):
                        o = 16 * h
                        x0v = xbuf[b, e, pl.ds(o, 16)]
                        x1x = xbuf[b, e, pl.ds(32 + o, 16)]
                        x1y = xbuf[b, e, pl.ds(64 + o, 16)]
                        x1z = xbuf[b, e, pl.ds(96 + o, 16)]
                        f000 = fnbuf[b, e, pl.ds(o, 16)]
                        f011 = fnbuf[b, e, pl.ds(32 + o, 16)]
                        f101 = fnbuf[b, e, pl.ds(64 + o, 16)]
                        f110 = fnbuf[b, e, pl.ds(96 + o, 16)]
                        xdu = x1x * ux + x1y * uy + x1z * uz
                        t = x0v * f011
                        mbuf[b, e, pl.ds(o, 16)] = x0v * f000 + f110 * xdu
                        mbuf[b, e, pl.ds(32 + o, 16)] = t * ux + x1x * f101
                        mbuf[b, e, pl.ds(64 + o, 16)] = t * uy + x1y * f101
                        mbuf[b, e, pl.ds(96 + o, 16)] = t * uz + x1z * f101

            def step(k, b, q):
                @pl.when(k >= 2)
                def _():
                    wait_sc(k - 2, b, (q + 2) % 4)
                wait_in(k, b, q)
                compute(k, b)
                wait_ii(k, q)
                scat(k, b, q)

                @pl.when(k + 2 <= _CPTH - 1)
                def _():
                    wait_jj(k + 2, (q + 2) % 4)
                    issue(k + 2, b, (q + 2) % 4)
                    issue_ii(k + 2, (q + 2) % 4)

                @pl.when(k + 3 <= _CPTH - 1)
                def _():
                    issue_jj(k + 3, (q + 3) % 4)

            if carry:
                pltpu.sync_copy(init_hbm.at[g, pl.ds(rows0, _RPT)],
                                acc.at[pl.ds(rows0, _RPT)])
            else:
                pltpu.sync_copy(init_hbm, acc.at[pl.ds(rows0, _RPT)])
            plsc.subcore_barrier()
            issue_jj(0, 0)
            issue_jj(1, 1)
            issue_jj(2, 2)
            issue_ii(0, 0)
            issue_ii(1, 1)
            wait_jj(0, 0)
            issue(0, 0, 0)
            wait_jj(1, 1)
            issue(1, 1, 1)

            @pl.loop(0, _CPTH - 1, step=4)
            def _(k):
                step(k, 0, 0)
                step(k + 1, 1, 1)
                step(k + 2, 0, 2)
                step(k + 3, 1, 3)

            step(_CPTH - 1, 0, 0)
            wait_sc(_CPTH - 2, 1, 3)
            wait_sc(_CPTH - 1, 0, 0)
            plsc.subcore_barrier()
            pltpu.sync_copy(acc.at[pl.ds(rows0, _RPT)],
                            out_hbm.at[g, pl.ds(rows0, _RPT)])
            plsc.subcore_barrier()

    return _agg


_agg_first = _make_agg(0, False)
_agg_second = _make_agg(_EH, True)


_NB = 400


def _final_body(out_ref, w0_ref, b0_ref, w1_ref,
                aw0_ref, ab0_ref, aw1_ref, ab1_ref,
                y0_ref, yx_ref, yy_ref, yz_ref):
    t = out_ref[...]

    def sec(s):
        return (t[:, :, s * _GC:(s + 1) * _GC]
                .transpose(1, 0, 2).reshape(_NB, _C))

    dn = (((1,), (1,)), ((), ()))
    kw = dict(preferred_element_type=jnp.float32,
              precision=lax.Precision.HIGHEST)
    h0 = lax.dot_general(sec(0), w0_ref[...], dn, **kw) + b0_ref[...]
    hx = lax.dot_general(sec(1), w1_ref[...], dn, **kw)
    hy = lax.dot_general(sec(2), w1_ref[...], dn, **kw)
    hz = lax.dot_general(sec(3), w1_ref[...], dn, **kw)
    y0_ref[...] = h0 * jax.nn.sigmoid(aw0_ref[...] * h0 + ab0_ref[...])
    nrm2 = hx * hx + hy * hy + hz * hz + 1e-12
    nrm = nrm2 * lax.rsqrt(nrm2)
    a = aw1_ref[...] * nrm + ab1_ref[...]
    gate = a * jax.nn.sigmoid(a)
    yx_ref[...] = hx * gate
    yy_ref[...] = hy * gate
    yz_ref[...] = hz * gate


def _final(out, w0, b0, w1, aw0, ab0, aw1, ab1):
    row = pl.BlockSpec((_NB, _C), lambda i: (i, 0))
    full = pl.BlockSpec((_C, _C), lambda i: (0, 0))
    vec = pl.BlockSpec((1, _C), lambda i: (0, 0))
    return pl.pallas_call(
        _final_body,
        grid=(_N // _NB,),
        in_specs=[pl.BlockSpec((_NG, _NB, _C), lambda i: (0, i, 0)),
                  full, vec, full, vec, vec, vec, vec],
        out_specs=[row, row, row, row],
        out_shape=[jax.ShapeDtypeStruct((_N, _C), jnp.float32)] * 4,
    )(out, w0, b0, w1, aw0, ab0, aw1, ab1)


def kernel(x0, x1, coordinates, W_rbf_000, W_rbf_011, W_rbf_101, W_rbf_110,
           W_si0, b_si0, W_si1, act_w0, act_b0, act_w1, act_b1,
           edge_index, atomic_number):
    idx_i = edge_index[0]
    idx_j = edge_index[1]
    coords_flat = jnp.pad(coordinates, ((0, 0), (0, 1))).reshape(-1)

    x1t = x1.transpose(2, 0, 1)
    parts = jnp.stack([x0.reshape(_N, _NG, _GC),
                       x1t[0].reshape(_N, _NG, _GC),
                       x1t[1].reshape(_N, _NG, _GC),
                       x1t[2].reshape(_N, _NG, _GC)], axis=2)
    xp = parts.transpose(1, 0, 2, 3).reshape(_NG, _N, _C)
    rij = _rij_kernel(coords_flat, idx_i, idx_j).reshape(_E, 4)

    w_all = (jnp.stack([W_rbf_000, W_rbf_011, W_rbf_101, W_rbf_110], 0)
             .reshape(_NG, _NG, _GC, _NRBF)
             .transpose(1, 0, 2, 3)
             .reshape(_NG * _C, _NRBF))
    fn1, u1 = _edge_features(rij, w_all, 0)
    fn2, u2 = _edge_features(rij, w_all, 1)

    z = jnp.zeros((_RPT, _C), jnp.float32)
    out1 = _agg_first(xp, fn1, u1.reshape(-1), idx_i, idx_j, z)
    out = _agg_second(xp, fn2, u2.reshape(-1), idx_i, idx_j, out1)

    y0, yx, yy, yz = _final(
        out, W_si0, b_si0.reshape(1, _C), W_si1,
        act_w0.reshape(1, _C), act_b0.reshape(1, _C),
        act_w1.reshape(1, _C), act_b1.reshape(1, _C))
    y1 = jnp.stack([yx, yy, yz], axis=-1)
    return (y0, y1)
